# speedup vs baseline: 3.0536x; 3.0536x over previous
import functools

import jax
import jax.numpy as jnp
from jax import lax
from jax.experimental import pallas as pl
from jax.experimental.pallas import tpu as pltpu
from jax.experimental.pallas import tpu_sc as plsc

D = 128
NC, NS = 2, 16
NW = NC * NS
CH = 128


@functools.lru_cache(maxsize=None)
def _make_gather(B: int):
    BPW = B // NW
    NCHUNK = BPW // CH
    mesh = plsc.VectorSubcoreMesh(core_axis_name="c", subcore_axis_name="s")

    @functools.partial(
        pl.kernel,
        out_type=jax.ShapeDtypeStruct((B, D), jnp.float32),
        mesh=mesh,
        scratch_types=[
            pltpu.VMEM((NCHUNK, CH), jnp.int32),
            pltpu.VMEM((CH, D), jnp.float32),
            pltpu.SemaphoreType.DMA,
        ],
    )
    def gather_kernel(idx_hbm, table_hbm, out_hbm, idx_v, rows_v, gsem):
        wid = lax.axis_index("s") * NC + lax.axis_index("c")
        base = wid * BPW
        pltpu.sync_copy(idx_hbm.at[wid], idx_v)

        @pl.loop(0, NCHUNK)
        def _chunk(j):
            pltpu.async_copy(table_hbm.at[idx_v.at[j]], rows_v, gsem).wait()
            pltpu.sync_copy(rows_v, out_hbm.at[pl.ds(base + j * CH, CH)])

    return gather_kernel


def kernel(x, table):
    batch, hist = x.shape
    B = batch * hist
    xf = x.reshape(-1).astype(jnp.int32).reshape(NW, B // NW // CH, CH)
    out = _make_gather(B)(xf, table)
    return out.reshape(batch, hist, D)


# speedup vs baseline: 3.4533x; 1.1309x over previous
import functools

import jax
import jax.numpy as jnp
from jax import lax
from jax.experimental import pallas as pl
from jax.experimental.pallas import tpu as pltpu
from jax.experimental.pallas import tpu_sc as plsc

D = 128
NC, NS = 2, 16
NW = NC * NS
CH = 128


NBUF = 4


@functools.lru_cache(maxsize=None)
def _make_gather(B: int):
    BPW = B // NW
    NCHUNK = BPW // CH
    assert NCHUNK % NBUF == 0
    mesh = plsc.VectorSubcoreMesh(core_axis_name="c", subcore_axis_name="s")

    @functools.partial(
        pl.kernel,
        out_type=jax.ShapeDtypeStruct((B, D), jnp.float32),
        mesh=mesh,
        scratch_types=[
            pltpu.VMEM((NCHUNK, CH), jnp.int32),
            pltpu.VMEM((NBUF, CH, D), jnp.float32),
            pltpu.SemaphoreType.DMA((NBUF,)),
            pltpu.SemaphoreType.DMA((NBUF,)),
        ],
    )
    def gather_kernel(idx_hbm, table_hbm, out_hbm, idx_v, rows_v, gsem, wsem):
        wid = lax.axis_index("s") * NC + lax.axis_index("c")
        base = wid * BPW
        pltpu.sync_copy(idx_hbm.at[wid], idx_v)

        def start_gather(j, b):
            pltpu.async_copy(table_hbm.at[idx_v.at[j]], rows_v.at[b],
                             gsem.at[b])

        def wait_gather(b):
            pltpu.make_async_copy(table_hbm.at[idx_v.at[0]], rows_v.at[b],
                                  gsem.at[b]).wait()

        def start_write(j, b):
            pltpu.async_copy(rows_v.at[b], out_hbm.at[pl.ds(base + j * CH, CH)],
                             wsem.at[b])

        def wait_write(b):
            pltpu.make_async_copy(rows_v.at[b], out_hbm.at[pl.ds(base, CH)],
                                  wsem.at[b]).wait()

        for b in range(NBUF):
            start_gather(b, b)

        @pl.loop(0, NCHUNK // NBUF)
        def _group(g):
            for b in range(NBUF):
                j = g * NBUF + b
                wait_gather(b)
                start_write(j, b)
                bp = (b - 1) % NBUF
                jp_next = j - 1 + NBUF

                @pl.when(j >= 1)
                def _():
                    wait_write(bp)

                    @pl.when(jp_next < NCHUNK)
                    def _():
                        start_gather(jp_next, bp)

        wait_write((NCHUNK - 1) % NBUF)

    return gather_kernel


def kernel(x, table):
    batch, hist = x.shape
    B = batch * hist
    xf = x.reshape(-1).astype(jnp.int32).reshape(NW, B // NW // CH, CH)
    out = _make_gather(B)(xf, table)
    return out.reshape(batch, hist, D)


# speedup vs baseline: 3.4601x; 1.0020x over previous
import functools

import jax
import jax.numpy as jnp
from jax import lax
from jax.experimental import pallas as pl
from jax.experimental.pallas import tpu as pltpu
from jax.experimental.pallas import tpu_sc as plsc

D = 128
NC, NS = 2, 16
NW = NC * NS
CH = 128


NBUF = 5


@functools.lru_cache(maxsize=None)
def _make_gather(B: int):
    BPW = B // NW
    NCHUNK = BPW // CH
    assert NCHUNK % NBUF == 0
    mesh = plsc.VectorSubcoreMesh(core_axis_name="c", subcore_axis_name="s")

    @functools.partial(
        pl.kernel,
        out_type=jax.ShapeDtypeStruct((B, D), jnp.float32),
        mesh=mesh,
        scratch_types=[
            pltpu.VMEM((NCHUNK, CH), jnp.int32),
            pltpu.VMEM((NBUF, CH, D), jnp.float32),
            pltpu.SemaphoreType.DMA((NBUF,)),
            pltpu.SemaphoreType.DMA((NBUF,)),
        ],
    )
    def gather_kernel(idx_hbm, table_hbm, out_hbm, idx_v, rows_v, gsem, wsem):
        wid = lax.axis_index("s") * NC + lax.axis_index("c")
        base = wid * BPW
        pltpu.sync_copy(idx_hbm.at[wid], idx_v)

        def start_gather(j, b):
            pltpu.async_copy(table_hbm.at[idx_v.at[j]], rows_v.at[b],
                             gsem.at[b])

        def wait_gather(b):
            pltpu.make_async_copy(table_hbm.at[idx_v.at[0]], rows_v.at[b],
                                  gsem.at[b]).wait()

        def start_write(j, b):
            pltpu.async_copy(rows_v.at[b], out_hbm.at[pl.ds(base + j * CH, CH)],
                             wsem.at[b])

        def wait_write(b):
            pltpu.make_async_copy(rows_v.at[b], out_hbm.at[pl.ds(base, CH)],
                                  wsem.at[b]).wait()

        for b in range(NBUF):
            start_gather(b, b)

        @pl.loop(0, NCHUNK // NBUF)
        def _group(g):
            for b in range(NBUF):
                j = g * NBUF + b
                wait_gather(b)
                start_write(j, b)
                bp = (b - 1) % NBUF
                jp_next = j - 1 + NBUF

                @pl.when(j >= 1)
                def _():
                    wait_write(bp)

                    @pl.when(jp_next < NCHUNK)
                    def _():
                        start_gather(jp_next, bp)

        wait_write((NCHUNK - 1) % NBUF)

    return gather_kernel


def kernel(x, table):
    batch, hist = x.shape
    B = batch * hist
    xf = x.reshape(-1).astype(jnp.int32).reshape(NW, B // NW // CH, CH)
    out = _make_gather(B)(xf, table)
    return out.reshape(batch, hist, D)
